# Initial kernel scaffold; baseline (speedup 1.0000x reference)
import jax
import jax.numpy as jnp
from jax.experimental import pallas as pl


def kernel(X, Z):
    raise NotImplementedError("write your pallas kernel here")


# speedup vs baseline: 19.3099x; 19.3099x over previous
import functools

import jax
import jax.numpy as jnp
from jax import lax
from jax.experimental import pallas as pl
from jax.experimental.pallas import tpu as pltpu
from jax.experimental.pallas import tpu_sc as plsc

_K = 5
_EPS = 1e-7
_N = 4096
_DX = 128
_DZ = 64
_DZP = 128
_ROWS = 256
_CHUNK = 512
_LN10 = 2.302585092994046


def _topk_body(xr_ref, xfull_ref, idx_ref, lid_ref, dsq_ref):
    xr = xr_ref[...]
    sxr = jnp.sum(xr * xr, axis=1, keepdims=True)
    for c in range(_N // _CHUNK):
        xc = xfull_ref[c * _CHUNK:(c + 1) * _CHUNK, :]
        sxc = jnp.sum(xc * xc, axis=1)
        g = lax.dot_general(xr, xc, (((1,), (1,)), ((), ())),
                            preferred_element_type=jnp.float32)
        dsq = sxr + sxc[None, :] - 2.0 * g
        dsq_ref[:, c * _CHUNK:(c + 1) * _CHUNK] = jnp.maximum(dsq, 1e-12)
    col = lax.broadcasted_iota(jnp.int32, (_ROWS, _N), 1)
    vals, idxs = [], []
    for j in range(_K + 1):
        d = dsq_ref[...]
        m = jnp.min(d, axis=1, keepdims=True)
        am = jnp.min(jnp.where(d == m, col, jnp.int32(2 ** 30)),
                     axis=1, keepdims=True)
        if j > 0:
            vals.append(m)
            idxs.append(am)
        dsq_ref[...] = jnp.where(col == am, jnp.float32(jnp.inf), d)
    v = jnp.sqrt(jnp.concatenate(vals, axis=1)) + _EPS
    logs = jnp.log(v)
    lid = -(jnp.sum(logs, axis=1, keepdims=True)
            - _K * logs[:, _K - 1:_K]) / _LN10
    idx_ref[...] = jnp.concatenate(
        idxs + [jnp.zeros((_ROWS, 8 - _K), jnp.int32)], axis=1)
    lid_ref[...] = jnp.broadcast_to(lid, (_ROWS, 8))


def _lid_loss_body(z_ref, zg_ref, lidx_ref, out_ref):
    z = z_ref[...]
    sz = jnp.sum(z * z, axis=1, keepdims=True)
    logs = []
    for j in range(_K):
        zg = zg_ref[:, j * _DZP:j * _DZP + _DZ]
        dot = jnp.sum(z * zg, axis=1, keepdims=True)
        szg = jnp.sum(zg * zg, axis=1, keepdims=True)
        zd = jnp.sqrt(jnp.maximum(sz + szg - 2.0 * dot, 1e-12)) + _EPS
        logs.append(jnp.log(zd))
    total = logs[0]
    for j in range(1, _K):
        total = total + logs[j]
    lid_z = -(total - _K * logs[_K - 1]) / _LN10
    diff = lidx_ref[:, 0:1] - lid_z
    out_ref[...] = (jnp.sum(diff * diff) / (_N * _K * 10)).reshape(1, 1)


_NC, _NS = 2, 16
_NW = _NC * _NS
_B = _N * _K
_BPW = _B // _NW
_GCH = 128


@functools.cache
def _make_gather_rows():
    @functools.partial(
        pl.kernel,
        mesh=plsc.VectorSubcoreMesh(core_axis_name="c", subcore_axis_name="s"),
        out_type=jax.ShapeDtypeStruct((_B, _DZP), jnp.float32),
        scratch_types=[
            pltpu.VMEM((_BPW,), jnp.int32),
            pltpu.VMEM((_BPW, _DZP), jnp.float32),
            pltpu.SemaphoreType.DMA,
        ],
    )
    def _gather_rows(table_hbm, idx_hbm, out_hbm, idx_v, rows_v, sem):
        wid = lax.axis_index("s") * _NC + lax.axis_index("c")
        base = wid * _BPW
        pltpu.sync_copy(idx_hbm.at[pl.ds(base, _BPW)], idx_v)
        copies = []
        for k in range(_BPW // _GCH):
            copies.append(pltpu.async_copy(
                table_hbm.at[idx_v.at[pl.ds(k * _GCH, _GCH)]],
                rows_v.at[pl.ds(k * _GCH, _GCH)], sem))
        for cp in copies:
            cp.wait()
        pltpu.sync_copy(rows_v, out_hbm.at[pl.ds(base, _BPW)])

    return _gather_rows


def kernel(X, Z):
    idx8, lidx8 = pl.pallas_call(
        _topk_body,
        grid=(_N // _ROWS,),
        in_specs=[
            pl.BlockSpec((_ROWS, _DX), lambda i: (i, 0)),
            pl.BlockSpec((_N, _DX), lambda i: (0, 0)),
        ],
        out_specs=[
            pl.BlockSpec((_ROWS, 8), lambda i: (i, 0)),
            pl.BlockSpec((_ROWS, 8), lambda i: (i, 0)),
        ],
        out_shape=[
            jax.ShapeDtypeStruct((_N, 8), jnp.int32),
            jax.ShapeDtypeStruct((_N, 8), jnp.float32),
        ],
        scratch_shapes=[pltpu.VMEM((_ROWS, _N), jnp.float32)],
    )(X, X)
    idx_flat = idx8[:, :_K].reshape(-1)
    z_pad = jnp.concatenate(
        [Z, jnp.zeros((_N, _DZP - _DZ), jnp.float32)], axis=1)
    zg = _make_gather_rows()(z_pad, idx_flat).reshape(_N, _K * _DZP)
    loss = pl.pallas_call(
        _lid_loss_body,
        out_shape=jax.ShapeDtypeStruct((1, 1), jnp.float32),
    )(Z, zg, lidx8)
    return loss[0, 0]


# speedup vs baseline: 21.1117x; 1.0933x over previous
import functools

import jax
import jax.numpy as jnp
from jax import lax
from jax.experimental import pallas as pl
from jax.experimental.pallas import tpu as pltpu
from jax.experimental.pallas import tpu_sc as plsc

_K = 5
_EPS = 1e-7
_N = 4096
_DX = 128
_DZ = 64
_DZP = 128
_ROWS = 256
_CHUNK = 512
_LN10 = 2.302585092994046


def _topk_body(xr_ref, xfull_ref, idx_ref, lid_ref, dsq_ref):
    i = pl.program_id(0)
    xr = xr_ref[...]
    sxr = jnp.sum(xr * xr, axis=1, keepdims=True)
    for c in range(_N // _CHUNK):
        xc = xfull_ref[c * _CHUNK:(c + 1) * _CHUNK, :]
        sxc = jnp.sum(xc * xc, axis=1)
        g = lax.dot_general(xr, xc, (((1,), (1,)), ((), ())),
                            preferred_element_type=jnp.float32)
        dsq = sxr + sxc[None, :] - 2.0 * g
        rowg = lax.broadcasted_iota(jnp.int32, (_ROWS, _CHUNK), 0) + i * _ROWS
        colg = lax.broadcasted_iota(jnp.int32, (_ROWS, _CHUNK), 1) + c * _CHUNK
        dsq = jnp.where(rowg == colg, jnp.float32(jnp.inf),
                        jnp.maximum(dsq, 1e-12))
        dsq_ref[:, c * _CHUNK:(c + 1) * _CHUNK] = dsq
    colf = lax.broadcasted_iota(jnp.int32, (_ROWS, _N), 1).astype(jnp.float32)
    vals, idxs = [], []
    for j in range(_K):
        d = dsq_ref[...]
        m = jnp.min(d, axis=1, keepdims=True)
        am = jnp.min(jnp.where(d == m, colf, jnp.float32(_N)),
                     axis=1, keepdims=True)
        vals.append(m)
        idxs.append(am.astype(jnp.int32))
        if j < _K - 1:
            dsq_ref[...] = jnp.where(colf == am, jnp.float32(jnp.inf), d)
    v = jnp.sqrt(jnp.concatenate(vals, axis=1)) + _EPS
    logs = jnp.log(v)
    lid = -(jnp.sum(logs, axis=1, keepdims=True)
            - _K * logs[:, _K - 1:_K]) / _LN10
    idx_ref[...] = jnp.concatenate(
        idxs + [jnp.zeros((_ROWS, 8 - _K), jnp.int32)], axis=1)
    lid_ref[...] = jnp.broadcast_to(lid, (_ROWS, 8))


def _lid_loss_body(z_ref, zg_ref, lidx_ref, idx_ref, out_ref):
    z = z_ref[...]
    sz = jnp.sum(z * z, axis=1, keepdims=True)
    logs = []
    for j in range(_K):
        blk = zg_ref[:, j * _DZP:(j + 1) * _DZP]
        par = (idx_ref[:, j:j + 1] & 1) == 1
        zg = jnp.where(par, blk[:, _DZ:], blk[:, :_DZ])
        dot = jnp.sum(z * zg, axis=1, keepdims=True)
        szg = jnp.sum(zg * zg, axis=1, keepdims=True)
        zd = jnp.sqrt(jnp.maximum(sz + szg - 2.0 * dot, 1e-12)) + _EPS
        logs.append(jnp.log(zd))
    total = logs[0]
    for j in range(1, _K):
        total = total + logs[j]
    lid_z = -(total - _K * logs[_K - 1]) / _LN10
    diff = lidx_ref[:, 0:1] - lid_z
    out_ref[...] = (jnp.sum(diff * diff) / (_N * _K * 10)).reshape(1, 1)


_NC, _NS = 2, 16
_NW = _NC * _NS
_B = _N * _K
_BPW = _B // _NW
_GCH = 128


@functools.cache
def _make_gather_rows():
    @functools.partial(
        pl.kernel,
        mesh=plsc.VectorSubcoreMesh(core_axis_name="c", subcore_axis_name="s"),
        out_type=jax.ShapeDtypeStruct((_B, _DZP), jnp.float32),
        scratch_types=[
            pltpu.VMEM((_BPW,), jnp.int32),
            pltpu.VMEM((_BPW, _DZP), jnp.float32),
            pltpu.SemaphoreType.DMA,
        ],
    )
    def _gather_rows(table_hbm, idx_hbm, out_hbm, idx_v, rows_v, sem):
        wid = lax.axis_index("s") * _NC + lax.axis_index("c")
        base = wid * _BPW
        pltpu.sync_copy(idx_hbm.at[pl.ds(base, _BPW)], idx_v)
        copies = []
        for k in range(_BPW // _GCH):
            copies.append(pltpu.async_copy(
                table_hbm.at[idx_v.at[pl.ds(k * _GCH, _GCH)]],
                rows_v.at[pl.ds(k * _GCH, _GCH)], sem))
        for cp in copies:
            cp.wait()
        pltpu.sync_copy(rows_v, out_hbm.at[pl.ds(base, _BPW)])

    return _gather_rows


def kernel(X, Z):
    idx8, lidx8 = pl.pallas_call(
        _topk_body,
        grid=(_N // _ROWS,),
        in_specs=[
            pl.BlockSpec((_ROWS, _DX), lambda i: (i, 0)),
            pl.BlockSpec((_N, _DX), lambda i: (0, 0)),
        ],
        out_specs=[
            pl.BlockSpec((_ROWS, 8), lambda i: (i, 0)),
            pl.BlockSpec((_ROWS, 8), lambda i: (i, 0)),
        ],
        out_shape=[
            jax.ShapeDtypeStruct((_N, 8), jnp.int32),
            jax.ShapeDtypeStruct((_N, 8), jnp.float32),
        ],
        scratch_shapes=[pltpu.VMEM((_ROWS, _N), jnp.float32)],
    )(X, X)
    idx_half = (idx8[:, :_K] // 2).reshape(-1)
    z_packed = Z.reshape(_N // 2, 2 * _DZ)
    zg = _make_gather_rows()(z_packed, idx_half).reshape(_N, _K * _DZP)
    loss = pl.pallas_call(
        _lid_loss_body,
        out_shape=jax.ShapeDtypeStruct((1, 1), jnp.float32),
    )(Z, zg, lidx8, idx8)
    return loss[0, 0]


# speedup vs baseline: 22.0616x; 1.0450x over previous
import functools

import jax
import jax.numpy as jnp
from jax import lax
from jax.experimental import pallas as pl
from jax.experimental.pallas import tpu as pltpu
from jax.experimental.pallas import tpu_sc as plsc

_K = 5
_EPS = 1e-7
_N = 4096
_DX = 128
_DZ = 64
_DZP = 128
_ROWS = 256
_CHUNK = 512
_LN10 = 2.302585092994046


def _topk_body(xr_ref, xfull_ref, idx_ref, lid_ref, dsq_ref):
    i = pl.program_id(0)
    xr = xr_ref[...]
    sxr = jnp.sum(xr * xr, axis=1, keepdims=True)
    for c in range(_N // _CHUNK):
        xc = xfull_ref[c * _CHUNK:(c + 1) * _CHUNK, :]
        sxc = jnp.sum(xc * xc, axis=1)
        g = lax.dot_general(xr, xc, (((1,), (1,)), ((), ())),
                            preferred_element_type=jnp.float32)
        dsq = sxr + sxc[None, :] - 2.0 * g
        rowg = lax.broadcasted_iota(jnp.int32, (_ROWS, _CHUNK), 0) + i * _ROWS
        colg = lax.broadcasted_iota(jnp.int32, (_ROWS, _CHUNK), 1) + c * _CHUNK
        dsq = jnp.where(rowg == colg, jnp.float32(jnp.inf),
                        jnp.maximum(dsq, 1e-12))
        dsq_ref[:, c * _CHUNK:(c + 1) * _CHUNK] = dsq
    colf = lax.broadcasted_iota(jnp.int32, (_ROWS, _N), 1).astype(jnp.float32)
    vals, idxs = [], []
    for j in range(_K):
        d = dsq_ref[...]
        m = jnp.min(d, axis=1, keepdims=True)
        am = jnp.min(jnp.where(d == m, colf, jnp.float32(_N)),
                     axis=1, keepdims=True)
        vals.append(m)
        idxs.append(am.astype(jnp.int32))
        if j < _K - 1:
            dsq_ref[...] = jnp.where(colf == am, jnp.float32(jnp.inf), d)
    v = jnp.sqrt(jnp.concatenate(vals, axis=1)) + _EPS
    logs = jnp.log(v)
    lid = -(jnp.sum(logs, axis=1, keepdims=True)
            - _K * logs[:, _K - 1:_K]) / _LN10
    idx_ref[...] = jnp.concatenate(
        idxs + [jnp.zeros((_ROWS, 8 - _K), jnp.int32)], axis=1)
    lid_ref[...] = jnp.broadcast_to(lid, (_ROWS, 8))


def _lid_loss_body(z_ref, zg_ref, lidx_ref, idx_ref, out_ref):
    z = z_ref[...]
    sz = jnp.sum(z * z, axis=1, keepdims=True)
    z2 = jnp.concatenate([z, z], axis=1)
    left = lax.broadcasted_iota(jnp.int32, (_N, _DZP), 1) < _DZ
    logs = []
    for j in range(_K):
        blk = zg_ref[:, j * _DZP:(j + 1) * _DZP]
        par = (idx_ref[:, j:j + 1] & 1) == 1
        use = jnp.logical_xor(par, left)
        dot = jnp.sum(jnp.where(use, z2 * blk, 0.0), axis=1, keepdims=True)
        szg = jnp.sum(jnp.where(use, blk * blk, 0.0), axis=1, keepdims=True)
        zd = jnp.sqrt(jnp.maximum(sz + szg - 2.0 * dot, 1e-12)) + _EPS
        logs.append(jnp.log(zd))
    total = logs[0]
    for j in range(1, _K):
        total = total + logs[j]
    lid_z = -(total - _K * logs[_K - 1]) / _LN10
    diff = lidx_ref[:, 0:1] - lid_z
    out_ref[...] = (jnp.sum(diff * diff) / (_N * _K * 10)).reshape(1, 1)


_NC, _NS = 2, 16
_NW = _NC * _NS
_B = _N * _K
_BPW = _B // _NW
_GCH = 128


@functools.cache
def _make_gather_rows():
    @functools.partial(
        pl.kernel,
        mesh=plsc.VectorSubcoreMesh(core_axis_name="c", subcore_axis_name="s"),
        out_type=jax.ShapeDtypeStruct((_B, _DZP), jnp.float32),
        scratch_types=[
            pltpu.VMEM((_BPW,), jnp.int32),
            pltpu.VMEM((_BPW, _DZP), jnp.float32),
            pltpu.SemaphoreType.DMA,
        ],
    )
    def _gather_rows(table_hbm, idx_hbm, out_hbm, idx_v, rows_v, sem):
        wid = lax.axis_index("s") * _NC + lax.axis_index("c")
        base = wid * _BPW
        pltpu.sync_copy(idx_hbm.at[pl.ds(base, _BPW)], idx_v)
        copies = []
        for k in range(_BPW // _GCH):
            copies.append(pltpu.async_copy(
                table_hbm.at[idx_v.at[pl.ds(k * _GCH, _GCH)]],
                rows_v.at[pl.ds(k * _GCH, _GCH)], sem))
        for cp in copies:
            cp.wait()
        pltpu.sync_copy(rows_v, out_hbm.at[pl.ds(base, _BPW)])

    return _gather_rows


def kernel(X, Z):
    idx8, lidx8 = pl.pallas_call(
        _topk_body,
        grid=(_N // _ROWS,),
        in_specs=[
            pl.BlockSpec((_ROWS, _DX), lambda i: (i, 0)),
            pl.BlockSpec((_N, _DX), lambda i: (0, 0)),
        ],
        out_specs=[
            pl.BlockSpec((_ROWS, 8), lambda i: (i, 0)),
            pl.BlockSpec((_ROWS, 8), lambda i: (i, 0)),
        ],
        out_shape=[
            jax.ShapeDtypeStruct((_N, 8), jnp.int32),
            jax.ShapeDtypeStruct((_N, 8), jnp.float32),
        ],
        scratch_shapes=[pltpu.VMEM((_ROWS, _N), jnp.float32)],
    )(X, X)
    idx_half = (idx8[:, :_K] // 2).reshape(-1)
    z_packed = Z.reshape(_N // 2, 2 * _DZ)
    zg = _make_gather_rows()(z_packed, idx_half).reshape(_N, _K * _DZP)
    loss = pl.pallas_call(
        _lid_loss_body,
        out_shape=jax.ShapeDtypeStruct((1, 1), jnp.float32),
    )(Z, zg, lidx8, idx8)
    return loss[0, 0]


# speedup vs baseline: 23.3167x; 1.0569x over previous
import functools

import jax
import jax.numpy as jnp
from jax import lax
from jax.experimental import pallas as pl
from jax.experimental.pallas import tpu as pltpu
from jax.experimental.pallas import tpu_sc as plsc

_K = 5
_EPS = 1e-7
_N = 4096
_DX = 128
_DZ = 64
_DZP = 128
_ROWS = 512
_CHUNK = 512
_LN10 = 2.302585092994046


def _topk_body(xr_ref, xfull_ref, idx_ref, lid_ref, dsq_ref):
    i = pl.program_id(0)
    xr = xr_ref[...]
    sxr = jnp.sum(xr * xr, axis=1, keepdims=True)
    for c in range(_N // _CHUNK):
        xc = xfull_ref[c * _CHUNK:(c + 1) * _CHUNK, :]
        sxc = jnp.sum(xc * xc, axis=1)
        g = lax.dot_general(xr, xc, (((1,), (1,)), ((), ())),
                            preferred_element_type=jnp.float32)
        dsq = sxr + sxc[None, :] - 2.0 * g
        rowg = lax.broadcasted_iota(jnp.int32, (_ROWS, _CHUNK), 0) + i * _ROWS
        colg = lax.broadcasted_iota(jnp.int32, (_ROWS, _CHUNK), 1) + c * _CHUNK
        dsq = jnp.where(rowg == colg, jnp.float32(jnp.inf),
                        jnp.maximum(dsq, 1e-12))
        dsq_ref[:, c * _CHUNK:(c + 1) * _CHUNK] = dsq
    colf = lax.broadcasted_iota(jnp.int32, (_ROWS, _N), 1).astype(jnp.float32)
    vals, idxs = [], []
    for j in range(_K):
        d = dsq_ref[...]
        m = jnp.min(d, axis=1, keepdims=True)
        am = jnp.min(jnp.where(d == m, colf, jnp.float32(_N)),
                     axis=1, keepdims=True)
        vals.append(m)
        idxs.append(am.astype(jnp.int32))
        if j < _K - 1:
            dsq_ref[...] = jnp.where(colf == am, jnp.float32(jnp.inf), d)
    v = jnp.sqrt(jnp.concatenate(vals, axis=1)) + _EPS
    logs = jnp.log(v)
    lid = -(jnp.sum(logs, axis=1, keepdims=True)
            - _K * logs[:, _K - 1:_K]) / _LN10
    idx_ref[...] = jnp.concatenate(
        idxs + [jnp.zeros((_ROWS, 8 - _K), jnp.int32)], axis=1)
    lid_ref[...] = jnp.broadcast_to(lid, (_ROWS, 8))


def _lid_loss_body(z_ref, zg_ref, lidx_ref, idx_ref, out_ref):
    z = z_ref[...]
    sz = jnp.sum(z * z, axis=1, keepdims=True)
    z2 = jnp.concatenate([z, z], axis=1)
    left = lax.broadcasted_iota(jnp.int32, (_N, _DZP), 1) < _DZ
    logs = []
    for j in range(_K):
        blk = zg_ref[:, j * _DZP:(j + 1) * _DZP]
        par = (idx_ref[:, j:j + 1] & 1) == 1
        use = jnp.logical_xor(par, left)
        dot = jnp.sum(jnp.where(use, z2 * blk, 0.0), axis=1, keepdims=True)
        szg = jnp.sum(jnp.where(use, blk * blk, 0.0), axis=1, keepdims=True)
        zd = jnp.sqrt(jnp.maximum(sz + szg - 2.0 * dot, 1e-12)) + _EPS
        logs.append(jnp.log(zd))
    total = logs[0]
    for j in range(1, _K):
        total = total + logs[j]
    lid_z = -(total - _K * logs[_K - 1]) / _LN10
    diff = lidx_ref[:, 0:1] - lid_z
    out_ref[...] = (jnp.sum(diff * diff) / (_N * _K * 10)).reshape(1, 1)


_NC, _NS = 2, 16
_NW = _NC * _NS
_B = _N * _K
_BPW = _B // _NW
_GCH = 128


@functools.cache
def _make_gather_rows():
    @functools.partial(
        pl.kernel,
        mesh=plsc.VectorSubcoreMesh(core_axis_name="c", subcore_axis_name="s"),
        out_type=jax.ShapeDtypeStruct((_B, _DZP), jnp.float32),
        scratch_types=[
            pltpu.VMEM((_BPW,), jnp.int32),
            pltpu.VMEM((_BPW, _DZP), jnp.float32),
            pltpu.SemaphoreType.DMA,
        ],
    )
    def _gather_rows(table_hbm, idx_hbm, out_hbm, idx_v, rows_v, sem):
        wid = lax.axis_index("s") * _NC + lax.axis_index("c")
        base = wid * _BPW
        pltpu.sync_copy(idx_hbm.at[pl.ds(base, _BPW)], idx_v)
        copies = []
        for k in range(_BPW // _GCH):
            copies.append(pltpu.async_copy(
                table_hbm.at[idx_v.at[pl.ds(k * _GCH, _GCH)]],
                rows_v.at[pl.ds(k * _GCH, _GCH)], sem))
        for cp in copies:
            cp.wait()
        pltpu.sync_copy(rows_v, out_hbm.at[pl.ds(base, _BPW)])

    return _gather_rows


def kernel(X, Z):
    idx8, lidx8 = pl.pallas_call(
        _topk_body,
        grid=(_N // _ROWS,),
        in_specs=[
            pl.BlockSpec((_ROWS, _DX), lambda i: (i, 0)),
            pl.BlockSpec((_N, _DX), lambda i: (0, 0)),
        ],
        out_specs=[
            pl.BlockSpec((_ROWS, 8), lambda i: (i, 0)),
            pl.BlockSpec((_ROWS, 8), lambda i: (i, 0)),
        ],
        out_shape=[
            jax.ShapeDtypeStruct((_N, 8), jnp.int32),
            jax.ShapeDtypeStruct((_N, 8), jnp.float32),
        ],
        scratch_shapes=[pltpu.VMEM((_ROWS, _N), jnp.float32)],
    )(X, X)
    idx_half = (idx8[:, :_K] // 2).reshape(-1)
    z_packed = Z.reshape(_N // 2, 2 * _DZ)
    zg = _make_gather_rows()(z_packed, idx_half).reshape(_N, _K * _DZP)
    loss = pl.pallas_call(
        _lid_loss_body,
        out_shape=jax.ShapeDtypeStruct((1, 1), jnp.float32),
    )(Z, zg, lidx8, idx8)
    return loss[0, 0]


# speedup vs baseline: 23.4780x; 1.0069x over previous
import functools

import jax
import jax.numpy as jnp
from jax import lax
from jax.experimental import pallas as pl
from jax.experimental.pallas import tpu as pltpu
from jax.experimental.pallas import tpu_sc as plsc

_K = 5
_EPS = 1e-7
_N = 4096
_DX = 128
_DZ = 64
_DZP = 128
_ROWS = 1024
_CHUNK = 512
_LN10 = 2.302585092994046


def _topk_body(xr_ref, xfull_ref, idx_ref, lid_ref, dsq_ref):
    i = pl.program_id(0)
    xr = xr_ref[...]
    sxr = jnp.sum(xr * xr, axis=1, keepdims=True)
    for c in range(_N // _CHUNK):
        xc = xfull_ref[c * _CHUNK:(c + 1) * _CHUNK, :]
        sxc = jnp.sum(xc * xc, axis=1)
        g = lax.dot_general(xr, xc, (((1,), (1,)), ((), ())),
                            preferred_element_type=jnp.float32)
        dsq = sxr + sxc[None, :] - 2.0 * g
        rowg = lax.broadcasted_iota(jnp.int32, (_ROWS, _CHUNK), 0) + i * _ROWS
        colg = lax.broadcasted_iota(jnp.int32, (_ROWS, _CHUNK), 1) + c * _CHUNK
        dsq = jnp.where(rowg == colg, jnp.float32(jnp.inf),
                        jnp.maximum(dsq, 1e-12))
        dsq_ref[:, c * _CHUNK:(c + 1) * _CHUNK] = dsq
    colf = lax.broadcasted_iota(jnp.int32, (_ROWS, _N), 1).astype(jnp.float32)
    vals, idxs = [], []
    for j in range(_K):
        d = dsq_ref[...]
        m = jnp.min(d, axis=1, keepdims=True)
        am = jnp.min(jnp.where(d == m, colf, jnp.float32(_N)),
                     axis=1, keepdims=True)
        vals.append(m)
        idxs.append(am.astype(jnp.int32))
        if j < _K - 1:
            dsq_ref[...] = jnp.where(colf == am, jnp.float32(jnp.inf), d)
    v = jnp.sqrt(jnp.concatenate(vals, axis=1)) + _EPS
    logs = jnp.log(v)
    lid = -(jnp.sum(logs, axis=1, keepdims=True)
            - _K * logs[:, _K - 1:_K]) / _LN10
    idx_ref[...] = jnp.concatenate(
        idxs + [jnp.zeros((_ROWS, 8 - _K), jnp.int32)], axis=1)
    lid_ref[...] = jnp.broadcast_to(lid, (_ROWS, 8))


def _lid_loss_body(z_ref, zg_ref, lidx_ref, idx_ref, out_ref):
    z = z_ref[...]
    sz = jnp.sum(z * z, axis=1, keepdims=True)
    z2 = jnp.concatenate([z, z], axis=1)
    left = lax.broadcasted_iota(jnp.int32, (_N, _DZP), 1) < _DZ
    logs = []
    for j in range(_K):
        blk = zg_ref[:, j * _DZP:(j + 1) * _DZP]
        par = (idx_ref[:, j:j + 1] & 1) == 1
        use = jnp.logical_xor(par, left)
        dot = jnp.sum(jnp.where(use, z2 * blk, 0.0), axis=1, keepdims=True)
        szg = jnp.sum(jnp.where(use, blk * blk, 0.0), axis=1, keepdims=True)
        zd = jnp.sqrt(jnp.maximum(sz + szg - 2.0 * dot, 1e-12)) + _EPS
        logs.append(jnp.log(zd))
    total = logs[0]
    for j in range(1, _K):
        total = total + logs[j]
    lid_z = -(total - _K * logs[_K - 1]) / _LN10
    diff = lidx_ref[:, 0:1] - lid_z
    out_ref[...] = (jnp.sum(diff * diff) / (_N * _K * 10)).reshape(1, 1)


_NC, _NS = 2, 16
_NW = _NC * _NS
_B = _N * _K
_BPW = _B // _NW
_GCH = 128


@functools.cache
def _make_gather_rows():
    @functools.partial(
        pl.kernel,
        mesh=plsc.VectorSubcoreMesh(core_axis_name="c", subcore_axis_name="s"),
        out_type=jax.ShapeDtypeStruct((_B, _DZP), jnp.float32),
        scratch_types=[
            pltpu.VMEM((_BPW,), jnp.int32),
            pltpu.VMEM((_BPW, _DZP), jnp.float32),
            pltpu.SemaphoreType.DMA,
        ],
    )
    def _gather_rows(table_hbm, idx_hbm, out_hbm, idx_v, rows_v, sem):
        wid = lax.axis_index("s") * _NC + lax.axis_index("c")
        base = wid * _BPW
        pltpu.sync_copy(idx_hbm.at[pl.ds(base, _BPW)], idx_v)
        copies = []
        for k in range(_BPW // _GCH):
            copies.append(pltpu.async_copy(
                table_hbm.at[idx_v.at[pl.ds(k * _GCH, _GCH)]],
                rows_v.at[pl.ds(k * _GCH, _GCH)], sem))
        for cp in copies:
            cp.wait()
        pltpu.sync_copy(rows_v, out_hbm.at[pl.ds(base, _BPW)])

    return _gather_rows


def kernel(X, Z):
    idx8, lidx8 = pl.pallas_call(
        _topk_body,
        grid=(_N // _ROWS,),
        in_specs=[
            pl.BlockSpec((_ROWS, _DX), lambda i: (i, 0)),
            pl.BlockSpec((_N, _DX), lambda i: (0, 0)),
        ],
        out_specs=[
            pl.BlockSpec((_ROWS, 8), lambda i: (i, 0)),
            pl.BlockSpec((_ROWS, 8), lambda i: (i, 0)),
        ],
        out_shape=[
            jax.ShapeDtypeStruct((_N, 8), jnp.int32),
            jax.ShapeDtypeStruct((_N, 8), jnp.float32),
        ],
        scratch_shapes=[pltpu.VMEM((_ROWS, _N), jnp.float32)],
    )(X, X)
    idx_half = (idx8[:, :_K] // 2).reshape(-1)
    z_packed = Z.reshape(_N // 2, 2 * _DZ)
    zg = _make_gather_rows()(z_packed, idx_half).reshape(_N, _K * _DZP)
    loss = pl.pallas_call(
        _lid_loss_body,
        out_shape=jax.ShapeDtypeStruct((1, 1), jnp.float32),
    )(Z, zg, lidx8, idx8)
    return loss[0, 0]
